# baseline (speedup 1.0000x reference)
import functools

import jax
import jax.numpy as jnp
from jax import lax
from jax.experimental import pallas as pl
from jax.experimental.pallas import tpu as pltpu
from jax.experimental.pallas import tpu_sc as plsc

_N, _D, _H, _L, _NG, _E = 10000, 128, 128, 64, 64, 320000
_NP = 10240
_NC, _NS = 2, 16
_NTILE = _NC * _NS
_EPT = _E // _NTILE
_K = 50
_NCHUNK = _EPT // _K
_W = 40
_NW = _NCHUNK // _W
_ROWS_PT = _NP // _NS


def _mp_round(h, src_r, dst_r, zeros_blk):
    mesh = plsc.VectorSubcoreMesh(
        core_axis_name="c", subcore_axis_name="s",
        num_cores=_NC, num_subcores=_NS)

    @functools.partial(
        pl.kernel,
        mesh=mesh,
        out_type=jax.ShapeDtypeStruct((_NC, _NP, _H), jnp.float32),
        scratch_types=[
            pltpu.VMEM((_W, _K), jnp.int32),
            pltpu.VMEM((_W, _K), jnp.int32),
            pltpu.VMEM((_W, _K), jnp.int32),
            pltpu.VMEM((_W, _K), jnp.int32),
            pltpu.VMEM((_K, _H), jnp.float32),
            pltpu.VMEM((_K, _H), jnp.float32),
            pltpu.VMEM((_K, _H), jnp.float32),
            pltpu.VMEM((_K, _H), jnp.float32),
            pltpu.VMEM_SHARED((_NP, _H), jnp.float32),
            pltpu.SemaphoreType.DMA,
            pltpu.SemaphoreType.DMA,
            pltpu.SemaphoreType.DMA,
            pltpu.SemaphoreType.DMA,
            pltpu.SemaphoreType.DMA,
            pltpu.SemaphoreType.DMA,
            pltpu.SemaphoreType.DMA,
            pltpu.SemaphoreType.DMA,
            pltpu.SemaphoreType.DMA,
            pltpu.SemaphoreType.DMA,
        ],
    )
    def body(h_hbm, src_hbm, dst_hbm, z_hbm, out_hbm,
             si0, si1, di0, di1, b0, b1, b2, b3, agg_sh,
             g0, g1, g2, g3, s0, s1, s2, s3, i0, i1):
        c = lax.axis_index("c")
        s = lax.axis_index("s")
        wid = s * _NC + c
        base = wid * _NW
        bufs = (b0, b1, b2, b3)
        gsem = (g0, g1, g2, g3)
        ssem = (s0, s1, s2, s3)
        iwins = [(si0, di0, i0), (si1, di1, i1)]
        pltpu.async_copy(src_hbm.at[base], si0, i0)
        pltpu.async_copy(dst_hbm.at[base], di0, i0)
        pltpu.sync_copy(z_hbm, agg_sh.at[pl.ds(s * _ROWS_PT, _ROWS_PT)])
        plsc.subcore_barrier()

        for w in range(_NW):
            siw, diw, isw = iwins[w % 2]
            pltpu.make_async_copy(src_hbm.at[base], siw, isw).wait()
            pltpu.make_async_copy(dst_hbm.at[base], diw, isw).wait()
            if w + 1 < _NW:
                nsi, ndi, nisw = iwins[(w + 1) % 2]
                pltpu.async_copy(src_hbm.at[base + w + 1], nsi, nisw)
                pltpu.async_copy(dst_hbm.at[base + w + 1], ndi, nisw)

            def gath(j, k):
                pltpu.async_copy(h_hbm.at[siw.at[j]], bufs[k], gsem[k])

            def gwait(k):
                pltpu.make_async_copy(
                    h_hbm.at[siw.at[0]], bufs[k], gsem[k]).wait()

            def sscat(j, k):
                pltpu.async_copy(
                    bufs[k], agg_sh.at[diw.at[j]], ssem[k], add=True)

            def swait(k):
                pltpu.make_async_copy(
                    bufs[k], agg_sh.at[diw.at[0]], ssem[k]).wait()

            gath(0, 0)
            gath(1, 1)
            gwait(0)
            sscat(0, 0)
            gath(2, 2)
            gwait(1)
            sscat(1, 1)
            gath(3, 3)

            def quad(i, carry):
                for t in range(4):
                    j = 4 * i + 2 + t
                    k = (2 + t) % 4
                    gwait(k)
                    sscat(j, k)
                    swait(t)
                    gath(j + 2, t)
                return carry

            lax.fori_loop(0, (_W - 4) // 4, quad, 0)
            gwait(2)
            sscat(_W - 2, 2)
            gwait(3)
            sscat(_W - 1, 3)
            for k in range(4):
                swait(k)

        plsc.subcore_barrier()
        pltpu.sync_copy(agg_sh.at[pl.ds(s * _ROWS_PT, _ROWS_PT)],
                        out_hbm.at[c, pl.ds(s * _ROWS_PT, _ROWS_PT)])

    return body(h, src_r, dst_r, zeros_blk)


_BLK = 2048


def _mm_relu(x, W, b, out_dtype=jnp.float32):
    def body(x_ref, w_ref, b_ref, o_ref):
        acc = lax.dot_general(x_ref[...], w_ref[...],
                              (((1,), (1,)), ((), ())),
                              preferred_element_type=jnp.float32)
        o_ref[...] = jnp.maximum(acc + b_ref[...], 0.0).astype(out_dtype)

    return pl.pallas_call(
        body,
        grid=(_NP // _BLK,),
        in_specs=[
            pl.BlockSpec((_BLK, _D), lambda i: (i, 0)),
            pl.BlockSpec((_H, _D), lambda i: (0, 0)),
            pl.BlockSpec((1, _H), lambda i: (0, 0)),
        ],
        out_specs=pl.BlockSpec((_BLK, _H), lambda i: (i, 0)),
        out_shape=jax.ShapeDtypeStruct((_NP, _H), out_dtype),
    )(x, W, b.reshape(1, _H))


def _mm_relu_sum(p, W, b, out_dtype=jnp.float32):
    def body(p_ref, w_ref, b_ref, o_ref):
        a = (p_ref[0].astype(jnp.float32) + p_ref[1].astype(jnp.float32))
        acc = lax.dot_general(a, w_ref[...],
                              (((1,), (1,)), ((), ())),
                              preferred_element_type=jnp.float32)
        o_ref[...] = jnp.maximum(acc + b_ref[...], 0.0).astype(out_dtype)

    return pl.pallas_call(
        body,
        grid=(_NP // _BLK,),
        in_specs=[
            pl.BlockSpec((2, _BLK, _H), lambda i: (0, i, 0)),
            pl.BlockSpec((_H, _H), lambda i: (0, 0)),
            pl.BlockSpec((1, _H), lambda i: (0, 0)),
        ],
        out_specs=pl.BlockSpec((_BLK, _H), lambda i: (i, 0)),
        out_shape=jax.ShapeDtypeStruct((_NP, _H), out_dtype),
    )(p, W, b.reshape(1, _H))


def _mm_relu_sum_pool_heads(p, W, b, batch_row, Wm, bm, Wl, bl):
    nblk = _NP // _BLK

    def body(p_ref, w_ref, b_ref, br_ref, wm_ref, bm_ref, wl_ref, bl_ref,
             om_ref, ol_ref, acc_ref):
        i = pl.program_id(0)
        a = p_ref[0] + p_ref[1]
        h = jnp.maximum(
            lax.dot_general(a, w_ref[...], (((1,), (1,)), ((), ())),
                            preferred_element_type=jnp.float32)
            + b_ref[...], 0.0)
        gids = lax.broadcasted_iota(jnp.int32, (_NG, _BLK), 0)
        mask = jnp.where(br_ref[...] == gids, 1.0, 0.0)
        part = lax.dot_general(mask, h, (((1,), (0,)), ((), ())),
                               preferred_element_type=jnp.float32)

        @pl.when(i == 0)
        def _():
            acc_ref[...] = part

        @pl.when(i > 0)
        def _():
            acc_ref[...] += part

        @pl.when(i == nblk - 1)
        def _():
            pooled = acc_ref[...]
            om_ref[...] = lax.dot_general(
                pooled, wm_ref[...], (((1,), (1,)), ((), ())),
                preferred_element_type=jnp.float32) + bm_ref[...]
            ol_ref[...] = lax.dot_general(
                pooled, wl_ref[...], (((1,), (1,)), ((), ())),
                preferred_element_type=jnp.float32) + bl_ref[...]

    return pl.pallas_call(
        body,
        grid=(nblk,),
        in_specs=[
            pl.BlockSpec((2, _BLK, _H), lambda i: (0, i, 0)),
            pl.BlockSpec((_H, _H), lambda i: (0, 0)),
            pl.BlockSpec((1, _H), lambda i: (0, 0)),
            pl.BlockSpec((1, _BLK), lambda i: (0, i)),
            pl.BlockSpec((_L, _H), lambda i: (0, 0)),
            pl.BlockSpec((1, _L), lambda i: (0, 0)),
            pl.BlockSpec((_L, _H), lambda i: (0, 0)),
            pl.BlockSpec((1, _L), lambda i: (0, 0)),
        ],
        out_specs=[pl.BlockSpec((_NG, _L), lambda i: (0, 0)),
                   pl.BlockSpec((_NG, _L), lambda i: (0, 0))],
        out_shape=[jax.ShapeDtypeStruct((_NG, _L), jnp.float32),
                   jax.ShapeDtypeStruct((_NG, _L), jnp.float32)],
        scratch_shapes=[pltpu.VMEM((_NG, _H), jnp.float32)],
    )(p, W, b.reshape(1, _H), batch_row, Wm, bm.reshape(1, _L),
      Wl, bl.reshape(1, _L))


def kernel(x, edge_index, batch, W_in, b_in, W1, b1, W2, b2, W3, b3,
           W_mean, b_mean, W_logvar, b_logvar):
    src_r = edge_index[0].reshape(_NTILE * _NW, _W, _K)
    dst_r = edge_index[1].reshape(_NTILE * _NW, _W, _K)
    zeros_blk = jnp.zeros((_ROWS_PT, _H), dtype=jnp.float32)
    x_pad = jnp.pad(x, ((0, _NP - _N), (0, 0)))
    batch_row = jnp.pad(batch.astype(jnp.int32), (0, _NP - _N),
                        constant_values=_NG).reshape(1, _NP)

    h = _mm_relu(x_pad, W_in, b_in)
    for W, b in ((W1, b1), (W2, b2)):
        p = _mp_round(h, src_r, dst_r, zeros_blk)
        h = _mm_relu_sum(p, W, b)
    p = _mp_round(h, src_r, dst_r, zeros_blk)
    z_mean, z_logvar = _mm_relu_sum_pool_heads(
        p, W3, b3, batch_row, W_mean, b_mean, W_logvar, b_logvar)
    return (z_mean, z_logvar)


# speedup vs baseline: 1.1379x; 1.1379x over previous
import functools

import jax
import jax.numpy as jnp
from jax import lax
from jax.experimental import pallas as pl
from jax.experimental.pallas import tpu as pltpu
from jax.experimental.pallas import tpu_sc as plsc

_N, _D, _H, _L, _NG, _E = 10000, 128, 128, 64, 64, 320000
_NP = 10240
_NC, _NS = 2, 16
_NTILE = _NC * _NS
_EPT = _E // _NTILE
_K = 100
_NCHUNK = _EPT // _K
_W = 10
_NW = _NCHUNK // _W
_ROWS_PT = _NP // _NS


def _mp_round(h, src_r, dst_r, zeros_blk):
    mesh = plsc.VectorSubcoreMesh(
        core_axis_name="c", subcore_axis_name="s",
        num_cores=_NC, num_subcores=_NS)

    @functools.partial(
        pl.kernel,
        mesh=mesh,
        out_type=jax.ShapeDtypeStruct((_NC, _NP, _H), jnp.float32),
        scratch_types=[
            pltpu.VMEM((_W, _K), jnp.int32),
            pltpu.VMEM((_W, _K), jnp.int32),
            pltpu.VMEM((_W, _K), jnp.int32),
            pltpu.VMEM((_W, _K), jnp.int32),
            pltpu.VMEM((_K, _H), jnp.float32),
            pltpu.VMEM((_K, _H), jnp.float32),
            pltpu.VMEM((_K, _H), jnp.float32),
            pltpu.VMEM_SHARED((_NP, _H), jnp.float32),
            pltpu.SemaphoreType.DMA,
            pltpu.SemaphoreType.DMA,
            pltpu.SemaphoreType.DMA,
            pltpu.SemaphoreType.DMA,
            pltpu.SemaphoreType.DMA,
            pltpu.SemaphoreType.DMA,
            pltpu.SemaphoreType.DMA,
            pltpu.SemaphoreType.DMA,
        ],
    )
    def body(h_hbm, src_hbm, dst_hbm, z_hbm, out_hbm,
             si0, si1, di0, di1, b0, b1, b2, agg_sh,
             g0, g1, g2, s0, s1, s2, i0, i1):
        c = lax.axis_index("c")
        s = lax.axis_index("s")
        wid = s * _NC + c
        base = wid * _NW
        bufs = (b0, b1, b2)
        gsem = (g0, g1, g2)
        ssem = (s0, s1, s2)
        iwins = [(si0, di0, i0), (si1, di1, i1)]
        pltpu.async_copy(src_hbm.at[base], si0, i0)
        pltpu.async_copy(dst_hbm.at[base], di0, i0)
        pltpu.sync_copy(z_hbm, agg_sh.at[pl.ds(s * _ROWS_PT, _ROWS_PT)])
        plsc.subcore_barrier()

        for w in range(_NW):
            siw, diw, isw = iwins[w % 2]
            pltpu.make_async_copy(src_hbm.at[base], siw, isw).wait()
            pltpu.make_async_copy(dst_hbm.at[base], diw, isw).wait()
            if w + 1 < _NW:
                nsi, ndi, nisw = iwins[(w + 1) % 2]
                pltpu.async_copy(src_hbm.at[base + w + 1], nsi, nisw)
                pltpu.async_copy(dst_hbm.at[base + w + 1], ndi, nisw)

            def gath(j, k):
                pltpu.async_copy(h_hbm.at[siw.at[j]], bufs[k], gsem[k])

            def gwait(k):
                pltpu.make_async_copy(
                    h_hbm.at[siw.at[0]], bufs[k], gsem[k]).wait()

            def sscat(j, k):
                pltpu.async_copy(
                    bufs[k], agg_sh.at[diw.at[j]], ssem[k], add=True)

            def swait(k):
                pltpu.make_async_copy(
                    bufs[k], agg_sh.at[diw.at[0]], ssem[k]).wait()

            gath(0, 0)
            gath(1, 1)
            gwait(0)
            sscat(0, 0)
            gath(2, 2)

            def trio(i, carry):
                for t in range(3):
                    j = 3 * i + 1 + t
                    k = (1 + t) % 3
                    gwait(k)
                    sscat(j, k)
                    swait(t)
                    gath(j + 2, t)
                return carry

            lax.fori_loop(0, (_W - 7) // 3, trio, 0)
            for j in range(_W - 6, _W - 2):
                k = j % 3
                gwait(k)
                sscat(j, k)
                swait((j + 2) % 3)
                gath(j + 2, (j + 2) % 3)
            gwait((_W - 2) % 3)
            sscat(_W - 2, (_W - 2) % 3)
            gwait((_W - 1) % 3)
            sscat(_W - 1, (_W - 1) % 3)
            for k in range(3):
                swait(k)

        plsc.subcore_barrier()
        pltpu.sync_copy(agg_sh.at[pl.ds(s * _ROWS_PT, _ROWS_PT)],
                        out_hbm.at[c, pl.ds(s * _ROWS_PT, _ROWS_PT)])

    return body(h, src_r, dst_r, zeros_blk)


_BLK = 2048


def _mm_relu(x, W, b, out_dtype=jnp.float32):
    def body(x_ref, w_ref, b_ref, o_ref):
        acc = lax.dot_general(x_ref[...], w_ref[...],
                              (((1,), (1,)), ((), ())),
                              preferred_element_type=jnp.float32)
        o_ref[...] = jnp.maximum(acc + b_ref[...], 0.0).astype(out_dtype)

    return pl.pallas_call(
        body,
        grid=(_NP // _BLK,),
        in_specs=[
            pl.BlockSpec((_BLK, _D), lambda i: (i, 0)),
            pl.BlockSpec((_H, _D), lambda i: (0, 0)),
            pl.BlockSpec((1, _H), lambda i: (0, 0)),
        ],
        out_specs=pl.BlockSpec((_BLK, _H), lambda i: (i, 0)),
        out_shape=jax.ShapeDtypeStruct((_NP, _H), out_dtype),
    )(x, W, b.reshape(1, _H))


def _mm_relu_sum(p, W, b, out_dtype=jnp.float32):
    def body(p_ref, w_ref, b_ref, o_ref):
        a = (p_ref[0].astype(jnp.float32) + p_ref[1].astype(jnp.float32))
        acc = lax.dot_general(a, w_ref[...],
                              (((1,), (1,)), ((), ())),
                              preferred_element_type=jnp.float32)
        o_ref[...] = jnp.maximum(acc + b_ref[...], 0.0).astype(out_dtype)

    return pl.pallas_call(
        body,
        grid=(_NP // _BLK,),
        in_specs=[
            pl.BlockSpec((2, _BLK, _H), lambda i: (0, i, 0)),
            pl.BlockSpec((_H, _H), lambda i: (0, 0)),
            pl.BlockSpec((1, _H), lambda i: (0, 0)),
        ],
        out_specs=pl.BlockSpec((_BLK, _H), lambda i: (i, 0)),
        out_shape=jax.ShapeDtypeStruct((_NP, _H), out_dtype),
    )(p, W, b.reshape(1, _H))


def _mm_relu_sum_pool_heads(p, W, b, batch_row, Wm, bm, Wl, bl):
    nblk = _NP // _BLK

    def body(p_ref, w_ref, b_ref, br_ref, wm_ref, bm_ref, wl_ref, bl_ref,
             om_ref, ol_ref, acc_ref):
        i = pl.program_id(0)
        a = p_ref[0] + p_ref[1]
        h = jnp.maximum(
            lax.dot_general(a, w_ref[...], (((1,), (1,)), ((), ())),
                            preferred_element_type=jnp.float32)
            + b_ref[...], 0.0)
        gids = lax.broadcasted_iota(jnp.int32, (_NG, _BLK), 0)
        mask = jnp.where(br_ref[...] == gids, 1.0, 0.0)
        part = lax.dot_general(mask, h, (((1,), (0,)), ((), ())),
                               preferred_element_type=jnp.float32)

        @pl.when(i == 0)
        def _():
            acc_ref[...] = part

        @pl.when(i > 0)
        def _():
            acc_ref[...] += part

        @pl.when(i == nblk - 1)
        def _():
            pooled = acc_ref[...]
            om_ref[...] = lax.dot_general(
                pooled, wm_ref[...], (((1,), (1,)), ((), ())),
                preferred_element_type=jnp.float32) + bm_ref[...]
            ol_ref[...] = lax.dot_general(
                pooled, wl_ref[...], (((1,), (1,)), ((), ())),
                preferred_element_type=jnp.float32) + bl_ref[...]

    return pl.pallas_call(
        body,
        grid=(nblk,),
        in_specs=[
            pl.BlockSpec((2, _BLK, _H), lambda i: (0, i, 0)),
            pl.BlockSpec((_H, _H), lambda i: (0, 0)),
            pl.BlockSpec((1, _H), lambda i: (0, 0)),
            pl.BlockSpec((1, _BLK), lambda i: (0, i)),
            pl.BlockSpec((_L, _H), lambda i: (0, 0)),
            pl.BlockSpec((1, _L), lambda i: (0, 0)),
            pl.BlockSpec((_L, _H), lambda i: (0, 0)),
            pl.BlockSpec((1, _L), lambda i: (0, 0)),
        ],
        out_specs=[pl.BlockSpec((_NG, _L), lambda i: (0, 0)),
                   pl.BlockSpec((_NG, _L), lambda i: (0, 0))],
        out_shape=[jax.ShapeDtypeStruct((_NG, _L), jnp.float32),
                   jax.ShapeDtypeStruct((_NG, _L), jnp.float32)],
        scratch_shapes=[pltpu.VMEM((_NG, _H), jnp.float32)],
    )(p, W, b.reshape(1, _H), batch_row, Wm, bm.reshape(1, _L),
      Wl, bl.reshape(1, _L))


def kernel(x, edge_index, batch, W_in, b_in, W1, b1, W2, b2, W3, b3,
           W_mean, b_mean, W_logvar, b_logvar):
    src_r = edge_index[0].reshape(_NTILE * _NW, _W, _K)
    dst_r = edge_index[1].reshape(_NTILE * _NW, _W, _K)
    zeros_blk = jnp.zeros((_ROWS_PT, _H), dtype=jnp.float32)
    x_pad = jnp.pad(x, ((0, _NP - _N), (0, 0)))
    batch_row = jnp.pad(batch.astype(jnp.int32), (0, _NP - _N),
                        constant_values=_NG).reshape(1, _NP)

    h = _mm_relu(x_pad, W_in, b_in)
    for W, b in ((W1, b1), (W2, b2)):
        p = _mp_round(h, src_r, dst_r, zeros_blk)
        h = _mm_relu_sum(p, W, b)
    p = _mp_round(h, src_r, dst_r, zeros_blk)
    z_mean, z_logvar = _mm_relu_sum_pool_heads(
        p, W3, b3, batch_row, W_mean, b_mean, W_logvar, b_logvar)
    return (z_mean, z_logvar)
